# Initial kernel scaffold; baseline (speedup 1.0000x reference)
import jax
import jax.numpy as jnp
from jax.experimental import pallas as pl


def kernel(x, camera2lidar, camera_intrinsics, img_aug_matrix, lidar_aug_matrix):
    raise NotImplementedError("write your pallas kernel here")


# speedup vs baseline: 2.8275x; 2.8275x over previous
import functools

import jax
import jax.numpy as jnp
import numpy as np
from jax import lax
from jax.experimental import pallas as pl
from jax.experimental.pallas import tpu as pltpu
from jax.experimental.pallas import tpu_sc as plsc

B, N, C = 1, 6, 64
D, FH, FW = 40, 32, 88
NX0, NX1 = 128, 128
NP = B * N * D * FH * FW
NSEG = NX0 * NX1
NPAD = 128
ACC_R = NSEG + NPAD
CH = 32
PPW = NP // 16
CHUNK = 128
NCHUNK = PPW // CHUNK
RPT = ACC_R // 16

_DX = np.array([0.8, 0.8, 20.0], dtype=np.float32)
_BX = np.array([-50.8, -50.8, 0.0], dtype=np.float32)
_OFF = _BX - _DX / np.float32(2.0)


def _voxel_body(p_ref, out_ref):
    X = p_ref[0, 0]
    Y = p_ref[0, 1]
    Z = p_ref[0, 2]
    gx = ((X - _OFF[0]) / _DX[0]).astype(jnp.int32)
    gy = ((Y - _OFF[1]) / _DX[1]).astype(jnp.int32)
    gz = ((Z - _OFF[2]) / _DX[2]).astype(jnp.int32)
    inb = (gx >= 0) & (gx < NX0) & (gy >= 0) & (gy < NX1) & (gz >= 0) & (gz < 1)
    hh = lax.broadcasted_iota(jnp.int32, (FH, FW), 0)
    ww = lax.broadcasted_iota(jnp.int32, (FH, FW), 1)
    pad_row = NSEG + ((hh * FW + ww) & (NPAD - 1))
    lin = jnp.where(inb, gx * NX1 + gy, pad_row)
    out_ref[0] = lin


def _compute_lin(pts_t):
    return pl.pallas_call(
        _voxel_body,
        grid=(N * D,),
        in_specs=[pl.BlockSpec((1, 3, FH, FW), lambda i: (i, 0, 0, 0))],
        out_specs=pl.BlockSpec((1, FH, FW), lambda i: (i, 0, 0)),
        out_shape=jax.ShapeDtypeStruct((N * D, FH, FW), jnp.int32),
    )(pts_t)


NZCH = ACC_R // CHUNK
CW = 8


def _sc_body(xh, linh, zr, zc, oh, sums_o, cnt_o, idx_v, rows_v, ones_v, big_v, cbig_v, acc_sh, cnt_sh):
    cid = lax.axis_index("c")
    sid = lax.axis_index("s")
    pltpu.sync_copy(zr, big_v)
    pltpu.sync_copy(zc, cbig_v)
    pltpu.sync_copy(oh, ones_v)
    for sv in range(16):

        @pl.when(sid == sv)
        def _():
            pltpu.sync_copy(big_v, acc_sh.at[pl.ds(sv * RPT, RPT)])
            pltpu.sync_copy(cbig_v, cnt_sh.at[pl.ds(sv * RPT, RPT)])

    plsc.subcore_barrier()

    base = sid * PPW

    def step(g, carry):
        off = base + g * CHUNK
        pltpu.sync_copy(linh.at[pl.ds(off, CHUNK)], idx_v.at[0])
        pltpu.sync_copy(xh.at[pl.ds(off, CHUNK), cid], rows_v)
        pltpu.sync_copy(rows_v, acc_sh.at[idx_v.at[0]], add=True)
        pltpu.sync_copy(ones_v, cnt_sh.at[idx_v.at[0]], add=True)
        return carry

    lax.fori_loop(0, NCHUNK, step, 0)
    plsc.subcore_barrier()

    for sv in range(16):

        @pl.when(sid == sv)
        def _():
            pltpu.sync_copy(acc_sh.at[pl.ds(sv * RPT, RPT)], big_v)
            pltpu.sync_copy(big_v, sums_o.at[cid, pl.ds(sv * RPT, RPT)])
            pltpu.sync_copy(cnt_sh.at[pl.ds(sv * RPT, RPT)], cbig_v)
            pltpu.sync_copy(cbig_v, cnt_o.at[cid, pl.ds(sv * RPT, RPT)])


def _sc_scatter(xh, lin):
    mesh = plsc.VectorSubcoreMesh(core_axis_name="c", subcore_axis_name="s")
    zr = jnp.zeros((RPT, CH), jnp.float32)
    zc = jnp.zeros((RPT, CW), jnp.float32)
    oh = jnp.ones((CHUNK, CW), jnp.float32)
    f = pl.kernel(
        _sc_body,
        out_type=[
            jax.ShapeDtypeStruct((2, ACC_R, CH), jnp.float32),
            jax.ShapeDtypeStruct((2, ACC_R, CW), jnp.float32),
        ],
        mesh=mesh,
        compiler_params=pltpu.CompilerParams(use_tc_tiling_on_sc=False),
        scratch_types=[
            pltpu.VMEM((1, CHUNK), jnp.int32),
            pltpu.VMEM((CHUNK, CH), jnp.float32),
            pltpu.VMEM((CHUNK, CW), jnp.float32),
            pltpu.VMEM((RPT, CH), jnp.float32),
            pltpu.VMEM((RPT, CW), jnp.float32),
            pltpu.VMEM_SHARED((ACC_R, CH), jnp.float32),
            pltpu.VMEM_SHARED((ACC_R, CW), jnp.float32),
        ],
    )
    return f(xh, lin, zr, zc, oh)


def _combine_body(s_ref, c_ref, o_ref):
    s = jnp.concatenate([s_ref[0, 0], s_ref[1, 0]], axis=1)
    c = c_ref[0, 0, :, 0]
    m = s / jnp.maximum(c, 1.0)[:, None]
    o_ref[0] = m.T


def _combine(sums, cnt):
    sums4 = sums.reshape(2, ACC_R // NX1, NX1, CH)
    cnt4 = cnt.reshape(2, ACC_R // NX1, NX1, CW)
    return pl.pallas_call(
        _combine_body,
        grid=(NX0,),
        in_specs=[
            pl.BlockSpec((2, 1, NX1, CH), lambda i: (0, i, 0, 0)),
            pl.BlockSpec((1, 1, NX1, CW), lambda i: (0, i, 0, 0)),
        ],
        out_specs=pl.BlockSpec((1, C, NX1), lambda i: (i, 0, 0)),
        out_shape=jax.ShapeDtypeStruct((NX0, C, NX1), jnp.float32),
    )(sums4, cnt4)


def kernel(x, camera2lidar, camera_intrinsics, img_aug_matrix, lidar_aug_matrix):
    intrins = camera_intrinsics[..., :3, :3]
    post_rots = img_aug_matrix[..., :3, :3]
    post_trans = img_aug_matrix[..., :3, 3]
    c2l_rots = camera2lidar[..., :3, :3]
    c2l_trans = camera2lidar[..., :3, 3]
    extra_rots = lidar_aug_matrix[..., :3, :3]
    extra_trans = lidar_aug_matrix[..., :3, 3]

    ds = jnp.broadcast_to(jnp.arange(1.0, 41.0, 1.0, dtype=jnp.float32).reshape(-1, 1, 1), (D, FH, FW))
    az = jnp.broadcast_to(jnp.linspace(-1.9, 1.9, FW, dtype=jnp.float32).reshape(1, 1, FW), (D, FH, FW))
    el = jnp.broadcast_to(jnp.linspace(-0.8, 0.8, FH, dtype=jnp.float32).reshape(1, FH, 1), (D, FH, FW))
    xs = ds * jnp.cos(el) * jnp.sin(az)
    ys = ds * jnp.sin(el)
    zs = ds * jnp.cos(el) * jnp.cos(az)
    frustum = jnp.stack((xs, ys, zs), -1)

    pts = frustum[None, None] - post_trans[:, :, None, None, None, :]
    pts = jnp.einsum('bnij,bndhwj->bndhwi', jnp.linalg.inv(post_rots), pts)
    pts = jnp.concatenate([pts[..., :2] * pts[..., 2:3], pts[..., 2:3]], axis=-1)
    comb = jnp.einsum('bnij,bnjk->bnik', c2l_rots, jnp.linalg.inv(intrins))
    pts = jnp.einsum('bnij,bndhwj->bndhwi', comb, pts)
    pts = pts + c2l_trans[:, :, None, None, None, :]
    pts = jnp.einsum('bij,bndhwj->bndhwi', extra_rots, pts)
    pts = pts + extra_trans[:, None, None, None, None, :]

    pts_t = pts.reshape(N * D, FH, FW, 3).transpose(0, 3, 1, 2)
    lin = _compute_lin(pts_t).reshape(NP)
    xh = x.reshape(NP, 2, CH)
    sums, cnt = _sc_scatter(xh, lin)
    out = _combine(sums, cnt)
    return out.transpose(1, 0, 2).reshape(B, C, NX0, NX1)


# speedup vs baseline: 3.8839x; 1.3736x over previous
import functools

import jax
import jax.numpy as jnp
import numpy as np
from jax import lax
from jax.experimental import pallas as pl
from jax.experimental.pallas import tpu as pltpu
from jax.experimental.pallas import tpu_sc as plsc

B, N, C = 1, 6, 64
D, FH, FW = 40, 32, 88
NX0, NX1 = 128, 128
NP = B * N * D * FH * FW
NSEG = NX0 * NX1
NPAD = 128
ACC_R = NSEG + NPAD
CH = 32
PPW = NP // 16
CHUNK = 128
NCHUNK = PPW // CHUNK
RPT = ACC_R // 16

_DX = np.array([0.8, 0.8, 20.0], dtype=np.float32)
_BX = np.array([-50.8, -50.8, 0.0], dtype=np.float32)
_OFF = _BX - _DX / np.float32(2.0)


def _voxel_body(p_ref, out_ref):
    X = p_ref[0, 0]
    Y = p_ref[0, 1]
    Z = p_ref[0, 2]
    gx = ((X - _OFF[0]) / _DX[0]).astype(jnp.int32)
    gy = ((Y - _OFF[1]) / _DX[1]).astype(jnp.int32)
    gz = ((Z - _OFF[2]) / _DX[2]).astype(jnp.int32)
    inb = (gx >= 0) & (gx < NX0) & (gy >= 0) & (gy < NX1) & (gz >= 0) & (gz < 1)
    hh = lax.broadcasted_iota(jnp.int32, (FH, FW), 0)
    ww = lax.broadcasted_iota(jnp.int32, (FH, FW), 1)
    pad_row = NSEG + ((hh * FW + ww) & (NPAD - 1))
    lin = jnp.where(inb, gx * NX1 + gy, pad_row)
    out_ref[0] = lin


def _compute_lin(pts_t):
    return pl.pallas_call(
        _voxel_body,
        grid=(N * D,),
        in_specs=[pl.BlockSpec((1, 3, FH, FW), lambda i: (i, 0, 0, 0))],
        out_specs=pl.BlockSpec((1, FH, FW), lambda i: (i, 0, 0)),
        out_shape=jax.ShapeDtypeStruct((N * D, FH, FW), jnp.int32),
    )(pts_t)


NZCH = ACC_R // CHUNK
CW = 8


def _sc_body(xh, linh, zr, zc, oh, sums_o, cnt_o, idx2, rows2, ones_v, big_v, cbig_v,
             si0, si1, sr0, sr1, acc_sh, cnt_sh):
    cid = lax.axis_index("c")
    sid = lax.axis_index("s")
    pltpu.sync_copy(zr, big_v)
    pltpu.sync_copy(zc, cbig_v)
    pltpu.sync_copy(oh, ones_v)
    for sv in range(16):

        @pl.when(sid == sv)
        def _():
            pltpu.sync_copy(big_v, acc_sh.at[pl.ds(sv * RPT, RPT)])
            pltpu.sync_copy(cbig_v, cnt_sh.at[pl.ds(sv * RPT, RPT)])

    plsc.subcore_barrier()

    base = sid * PPW
    si = (si0, si1)
    sr = (sr0, sr1)

    def issue(g, b):
        off = base + g * CHUNK
        pltpu.async_copy(linh.at[pl.ds(off, CHUNK)], idx2.at[b], si[b])
        pltpu.async_copy(xh.at[pl.ds(off, CHUNK), cid], rows2.at[b], sr[b])

    def wait_scatter(g, b):
        off = base + g * CHUNK
        pltpu.make_async_copy(linh.at[pl.ds(off, CHUNK)], idx2.at[b], si[b]).wait()
        pltpu.make_async_copy(xh.at[pl.ds(off, CHUNK), cid], rows2.at[b], sr[b]).wait()
        pltpu.sync_copy(rows2.at[b], acc_sh.at[idx2.at[b]], add=True)
        pltpu.sync_copy(ones_v, cnt_sh.at[idx2.at[b]], add=True)

    issue(0, 0)
    issue(1, 1)

    @pl.loop(0, NCHUNK - 2, step=2)
    def _(g0):
        for b in range(2):
            g = g0 + b
            wait_scatter(g, b)
            issue(g + 2, b)

    for b in range(2):
        wait_scatter(NCHUNK - 2 + b, b)

    plsc.subcore_barrier()

    for sv in range(16):

        @pl.when(sid == sv)
        def _():
            pltpu.sync_copy(acc_sh.at[pl.ds(sv * RPT, RPT)], big_v)
            pltpu.sync_copy(big_v, sums_o.at[cid, pl.ds(sv * RPT, RPT)])
            pltpu.sync_copy(cnt_sh.at[pl.ds(sv * RPT, RPT)], cbig_v)
            pltpu.sync_copy(cbig_v, cnt_o.at[cid, pl.ds(sv * RPT, RPT)])


def _sc_scatter(xh, lin):
    mesh = plsc.VectorSubcoreMesh(core_axis_name="c", subcore_axis_name="s")
    zr = jnp.zeros((RPT, CH), jnp.float32)
    zc = jnp.zeros((RPT, CW), jnp.float32)
    oh = jnp.ones((CHUNK, CW), jnp.float32)
    f = pl.kernel(
        _sc_body,
        out_type=[
            jax.ShapeDtypeStruct((2, ACC_R, CH), jnp.float32),
            jax.ShapeDtypeStruct((2, ACC_R, CW), jnp.float32),
        ],
        mesh=mesh,
        compiler_params=pltpu.CompilerParams(use_tc_tiling_on_sc=False),
        scratch_types=[
            pltpu.VMEM((2, CHUNK), jnp.int32),
            pltpu.VMEM((2, CHUNK, CH), jnp.float32),
            pltpu.VMEM((CHUNK, CW), jnp.float32),
            pltpu.VMEM((RPT, CH), jnp.float32),
            pltpu.VMEM((RPT, CW), jnp.float32),
            pltpu.SemaphoreType.DMA,
            pltpu.SemaphoreType.DMA,
            pltpu.SemaphoreType.DMA,
            pltpu.SemaphoreType.DMA,
            pltpu.VMEM_SHARED((ACC_R, CH), jnp.float32),
            pltpu.VMEM_SHARED((ACC_R, CW), jnp.float32),
        ],
    )
    return f(xh, lin, zr, zc, oh)


def _combine_body(s_ref, c_ref, o_ref):
    s = jnp.concatenate([s_ref[0, 0], s_ref[1, 0]], axis=1)
    c = c_ref[0, 0, :, 0]
    m = s / jnp.maximum(c, 1.0)[:, None]
    o_ref[0] = m.T


def _combine(sums, cnt):
    sums4 = sums.reshape(2, ACC_R // NX1, NX1, CH)
    cnt4 = cnt.reshape(2, ACC_R // NX1, NX1, CW)
    return pl.pallas_call(
        _combine_body,
        grid=(NX0,),
        in_specs=[
            pl.BlockSpec((2, 1, NX1, CH), lambda i: (0, i, 0, 0)),
            pl.BlockSpec((1, 1, NX1, CW), lambda i: (0, i, 0, 0)),
        ],
        out_specs=pl.BlockSpec((1, C, NX1), lambda i: (i, 0, 0)),
        out_shape=jax.ShapeDtypeStruct((NX0, C, NX1), jnp.float32),
    )(sums4, cnt4)


def kernel(x, camera2lidar, camera_intrinsics, img_aug_matrix, lidar_aug_matrix):
    intrins = camera_intrinsics[..., :3, :3]
    post_rots = img_aug_matrix[..., :3, :3]
    post_trans = img_aug_matrix[..., :3, 3]
    c2l_rots = camera2lidar[..., :3, :3]
    c2l_trans = camera2lidar[..., :3, 3]
    extra_rots = lidar_aug_matrix[..., :3, :3]
    extra_trans = lidar_aug_matrix[..., :3, 3]

    ds = jnp.broadcast_to(jnp.arange(1.0, 41.0, 1.0, dtype=jnp.float32).reshape(-1, 1, 1), (D, FH, FW))
    az = jnp.broadcast_to(jnp.linspace(-1.9, 1.9, FW, dtype=jnp.float32).reshape(1, 1, FW), (D, FH, FW))
    el = jnp.broadcast_to(jnp.linspace(-0.8, 0.8, FH, dtype=jnp.float32).reshape(1, FH, 1), (D, FH, FW))
    xs = ds * jnp.cos(el) * jnp.sin(az)
    ys = ds * jnp.sin(el)
    zs = ds * jnp.cos(el) * jnp.cos(az)
    frustum = jnp.stack((xs, ys, zs), -1)

    pts = frustum[None, None] - post_trans[:, :, None, None, None, :]
    pts = jnp.einsum('bnij,bndhwj->bndhwi', jnp.linalg.inv(post_rots), pts)
    pts = jnp.concatenate([pts[..., :2] * pts[..., 2:3], pts[..., 2:3]], axis=-1)
    comb = jnp.einsum('bnij,bnjk->bnik', c2l_rots, jnp.linalg.inv(intrins))
    pts = jnp.einsum('bnij,bndhwj->bndhwi', comb, pts)
    pts = pts + c2l_trans[:, :, None, None, None, :]
    pts = jnp.einsum('bij,bndhwj->bndhwi', extra_rots, pts)
    pts = pts + extra_trans[:, None, None, None, None, :]

    pts_t = pts.reshape(N * D, FH, FW, 3).transpose(0, 3, 1, 2)
    lin = _compute_lin(pts_t).reshape(NP)
    xh = x.reshape(NP, 2, CH)
    sums, cnt = _sc_scatter(xh, lin)
    out = _combine(sums, cnt)
    return out.transpose(1, 0, 2).reshape(B, C, NX0, NX1)
